# baseline (speedup 1.0000x reference)
import functools

import jax
import jax.numpy as jnp
from jax import lax
from jax.experimental import pallas as pl
from jax.experimental.pallas import tpu as pltpu
from jax.experimental.pallas import tpu_sc as plsc

B = 1024
S = 200
HIDDEN = 128
EPS = 1e-12

N_TOK = B * S
NC, NS = 2, 16
NW = NC * NS
IDS_COLS = 128
IDS_ROWS = N_TOK // IDS_COLS
CHUNK_ROWS = 5
CHUNK_TOK = CHUNK_ROWS * IDS_COLS
SLAB_TOKS = (102400, 102400)
N_SLAB = len(SLAB_TOKS)


def _sc_gather(table, ids3, slab_tok):
    n_chunks = slab_tok // CHUNK_TOK
    n_iters = (n_chunks + NW - 1) // NW
    mesh = plsc.VectorSubcoreMesh(core_axis_name="c", subcore_axis_name="s")

    @functools.partial(
        pl.kernel,
        mesh=mesh,
        out_type=jax.ShapeDtypeStruct((slab_tok, HIDDEN), jnp.float32),
        scratch_types=[
            pltpu.VMEM((CHUNK_ROWS, IDS_COLS), jnp.int32),
            pltpu.VMEM((CHUNK_TOK, HIDDEN), jnp.float32),
            pltpu.SemaphoreType.DMA,
        ],
    )
    def gather_kernel(table_hbm, ids_hbm, out_hbm, idx_v, rows_v, sem):
        wid = lax.axis_index("s") * NC + lax.axis_index("c")

        def body(i, carry):
            c = wid + i * NW

            @pl.when(c < n_chunks)
            def _():
                pltpu.sync_copy(ids_hbm.at[c], idx_v)
                handles = []
                for j in range(CHUNK_ROWS):
                    handles.append(pltpu.async_copy(
                        table_hbm.at[idx_v.at[j]],
                        rows_v.at[pl.ds(j * IDS_COLS, IDS_COLS)],
                        sem))
                for h in handles:
                    h.wait()
                pltpu.sync_copy(
                    rows_v, out_hbm.at[pl.ds(c * CHUNK_TOK, CHUNK_TOK)])

            return carry

        lax.fori_loop(0, n_iters, body, 0)

    return gather_kernel(table, ids3)


TOK_BLK = 12800
TT_ROWS = TOK_BLK // 128


def _ln_body(g_ref, tt_ref, pos_ref, type_ref, gam_ref, bet_ref, o_ref):
    x = g_ref[...] + pos_ref[...]
    t0 = type_ref[0:1, :]
    dt = type_ref[1:2, :] - t0
    t2t = jnp.transpose(tt_ref[0])
    term = jnp.concatenate(
        [t2t[:, r:r + 1] * dt for r in range(TT_ROWS)], axis=0)
    x = x + t0 + term
    inv_h = jnp.float32(1.0 / HIDDEN)
    mean = jnp.sum(x, axis=1, keepdims=True) * inv_h
    ex2 = jnp.sum(x * x, axis=1, keepdims=True) * inv_h
    var = ex2 - mean * mean
    inv_std = lax.rsqrt(var + EPS)
    a = inv_std * gam_ref[...]
    o_ref[...] = (x - mean) * a + bet_ref[...]


def _ln_body_alias(g_ref, tt_ref, pos_ref, type_ref, gam_ref, bet_ref,
                   prev_ref, o_ref):
    _ln_body(g_ref, tt_ref, pos_ref, type_ref, gam_ref, bet_ref, o_ref)


def _tc_add_ln(blk_base, gathered, tt_packed, pos_tile, W_type, gamma, beta,
               prev_out):
    n_blk = gathered.shape[0] // TOK_BLK
    in_specs = [
        pl.BlockSpec((TOK_BLK, HIDDEN), lambda i: (i, 0)),
        pl.BlockSpec((1, TT_ROWS, 128), lambda i, b=blk_base: (b + i, 0, 0)),
        pl.BlockSpec((TOK_BLK, HIDDEN), lambda i: (0, 0)),
        pl.BlockSpec((2, HIDDEN), lambda i: (0, 0)),
        pl.BlockSpec((1, HIDDEN), lambda i: (0, 0)),
        pl.BlockSpec((1, HIDDEN), lambda i: (0, 0)),
    ]
    args = [gathered, tt_packed, pos_tile, W_type, gamma, beta]
    kwargs = {}
    body = _ln_body
    if prev_out is not None:
        in_specs.append(pl.BlockSpec(memory_space=pl.ANY))
        args.append(prev_out)
        kwargs["input_output_aliases"] = {6: 0}
        body = _ln_body_alias
    return pl.pallas_call(
        body,
        grid=(n_blk,),
        in_specs=in_specs,
        out_specs=pl.BlockSpec((TOK_BLK, HIDDEN),
                               lambda i, b=blk_base: (b + i, 0)),
        out_shape=jax.ShapeDtypeStruct((N_TOK, HIDDEN), jnp.float32),
        **kwargs,
    )(*args)


def kernel(input_ids, token_type_ids, W_word, W_pos, W_type, gamma, beta):
    ids = input_ids.astype(jnp.int32).reshape(-1)
    tt_packed = token_type_ids.astype(jnp.float32).reshape(
        N_TOK // TOK_BLK, TT_ROWS, 128)
    pos_tile = jnp.tile(W_pos[:S], (TOK_BLK // S, 1))
    gam = gamma.reshape(1, HIDDEN)
    bet = beta.reshape(1, HIDDEN)

    slabs = []
    tok0 = 0
    for st in SLAB_TOKS:
        ids3 = ids[tok0:tok0 + st].reshape(st // CHUNK_TOK, CHUNK_ROWS,
                                           IDS_COLS)
        slabs.append(_sc_gather(W_word, ids3, st))
        tok0 += st

    out = None
    tok0 = 0
    for s, st in enumerate(SLAB_TOKS):
        out = _tc_add_ln(tok0 // TOK_BLK, slabs[s], tt_packed, pos_tile,
                         W_type, gam, bet, out)
        tok0 += st
    return out.reshape(B, S, HIDDEN)


# speedup vs baseline: 1.0094x; 1.0094x over previous
import functools

import jax
import jax.numpy as jnp
from jax import lax
from jax.experimental import pallas as pl
from jax.experimental.pallas import tpu as pltpu
from jax.experimental.pallas import tpu_sc as plsc

B = 1024
S = 200
HIDDEN = 128
EPS = 1e-12

N_TOK = B * S
NC, NS = 2, 16
NW = NC * NS
IDS_COLS = 128
IDS_ROWS = N_TOK // IDS_COLS
CHUNK_ROWS = 5
CHUNK_TOK = CHUNK_ROWS * IDS_COLS
SLAB_TOKS = (102400, 102400)
N_SLAB = len(SLAB_TOKS)


def _sc_gather(table, ids3, slab_tok):
    n_chunks = slab_tok // CHUNK_TOK
    n_iters = (n_chunks + NW - 1) // NW
    mesh = plsc.VectorSubcoreMesh(core_axis_name="c", subcore_axis_name="s")

    @functools.partial(
        pl.kernel,
        mesh=mesh,
        out_type=jax.ShapeDtypeStruct((slab_tok, HIDDEN), jnp.float32),
        scratch_types=[
            pltpu.VMEM((CHUNK_ROWS, IDS_COLS), jnp.int32),
            pltpu.VMEM((CHUNK_TOK, HIDDEN), jnp.float32),
            pltpu.SemaphoreType.DMA,
        ],
    )
    def gather_kernel(table_hbm, ids_hbm, out_hbm, idx_v, rows_v, sem):
        wid = lax.axis_index("s") * NC + lax.axis_index("c")

        def body(i, carry):
            c = wid + i * NW

            @pl.when(c < n_chunks)
            def _():
                pltpu.sync_copy(ids_hbm.at[c], idx_v)
                handles = []
                for j in range(CHUNK_ROWS):
                    handles.append(pltpu.async_copy(
                        table_hbm.at[idx_v.at[j]],
                        rows_v.at[pl.ds(j * IDS_COLS, IDS_COLS)],
                        sem))
                for h in handles:
                    h.wait()
                pltpu.sync_copy(
                    rows_v, out_hbm.at[pl.ds(c * CHUNK_TOK, CHUNK_TOK)])

            return carry

        lax.fori_loop(0, n_iters, body, 0)

    return gather_kernel(table, ids3)


TOK_BLK = 6400
TT_ROWS = TOK_BLK // 128


def _ln_body(g_ref, tt_ref, pos_ref, type_ref, gam_ref, bet_ref, o_ref):
    x = g_ref[...] + pos_ref[...]
    t0 = type_ref[0:1, :]
    dt = type_ref[1:2, :] - t0
    t2t = jnp.transpose(tt_ref[0])
    term = jnp.concatenate(
        [t2t[:, r:r + 1] * dt for r in range(TT_ROWS)], axis=0)
    x = x + t0 + term
    inv_h = jnp.float32(1.0 / HIDDEN)
    mean = jnp.sum(x, axis=1, keepdims=True) * inv_h
    ex2 = jnp.sum(x * x, axis=1, keepdims=True) * inv_h
    var = ex2 - mean * mean
    inv_std = lax.rsqrt(var + EPS)
    a = inv_std * gam_ref[...]
    o_ref[...] = (x - mean) * a + bet_ref[...]


def _ln_body_alias(g_ref, tt_ref, pos_ref, type_ref, gam_ref, bet_ref,
                   prev_ref, o_ref):
    _ln_body(g_ref, tt_ref, pos_ref, type_ref, gam_ref, bet_ref, o_ref)


def _tc_add_ln(blk_base, gathered, tt_packed, pos_tile, W_type, gamma, beta,
               prev_out):
    n_blk = gathered.shape[0] // TOK_BLK
    in_specs = [
        pl.BlockSpec((TOK_BLK, HIDDEN), lambda i: (i, 0)),
        pl.BlockSpec((1, TT_ROWS, 128), lambda i, b=blk_base: (b + i, 0, 0)),
        pl.BlockSpec((TOK_BLK, HIDDEN), lambda i: (0, 0)),
        pl.BlockSpec((2, HIDDEN), lambda i: (0, 0)),
        pl.BlockSpec((1, HIDDEN), lambda i: (0, 0)),
        pl.BlockSpec((1, HIDDEN), lambda i: (0, 0)),
    ]
    args = [gathered, tt_packed, pos_tile, W_type, gamma, beta]
    kwargs = {}
    body = _ln_body
    if prev_out is not None:
        in_specs.append(pl.BlockSpec(memory_space=pl.ANY))
        args.append(prev_out)
        kwargs["input_output_aliases"] = {6: 0}
        body = _ln_body_alias
    return pl.pallas_call(
        body,
        grid=(n_blk,),
        in_specs=in_specs,
        out_specs=pl.BlockSpec((TOK_BLK, HIDDEN),
                               lambda i, b=blk_base: (b + i, 0)),
        out_shape=jax.ShapeDtypeStruct((N_TOK, HIDDEN), jnp.float32),
        **kwargs,
    )(*args)


def kernel(input_ids, token_type_ids, W_word, W_pos, W_type, gamma, beta):
    ids = input_ids.astype(jnp.int32).reshape(-1)
    tt_packed = token_type_ids.astype(jnp.float32).reshape(
        N_TOK // TOK_BLK, TT_ROWS, 128)
    pos_tile = jnp.tile(W_pos[:S], (TOK_BLK // S, 1))
    gam = gamma.reshape(1, HIDDEN)
    bet = beta.reshape(1, HIDDEN)

    slabs = []
    tok0 = 0
    for st in SLAB_TOKS:
        ids3 = ids[tok0:tok0 + st].reshape(st // CHUNK_TOK, CHUNK_ROWS,
                                           IDS_COLS)
        slabs.append(_sc_gather(W_word, ids3, st))
        tok0 += st

    out = None
    tok0 = 0
    for s, st in enumerate(SLAB_TOKS):
        out = _tc_add_ln(tok0 // TOK_BLK, slabs[s], tt_packed, pos_tile,
                         W_type, gam, bet, out)
        tok0 += st
    return out.reshape(B, S, HIDDEN)
